# baseline (speedup 1.0000x reference)
import dataclasses
import functools

import jax
import jax.numpy as jnp
from jax import lax
from jax.experimental import pallas as pl
from jax.experimental.pallas import tpu as pltpu
from jax.experimental.pallas import tpu_sc as plsc

ROWS = 64
N = 32768
TOPK = 64
L = 16
NCHUNK = N // L
NB = 4096
NC = NB // L
NB3 = 256
NWORKERS = 32
ROWS_PER_W = ROWS // NWORKERS
UNROLL = 8
INT_MIN = -(2**31)


def _monotone(xi):
    return xi ^ (jnp.int32(0x7FFFFFFF) & (xi >> 31))


def _remap_chunk(xv, i):
    xx = xv[pl.ds(i, L)]
    return xx, _monotone(plsc.bitcast(xx, jnp.int32))


NSUB = 4


def _h4(hist, c):
    h = hist[pl.ds(c * L, L)]
    for s in range(1, NSUB):
        h = h + hist[pl.ds(s * NB + c * L, L)]
    return h


def _scan_fine_h(h, c, want):
    iota = lax.iota(jnp.int32, L)
    hr = lax.rev(h, (0,))
    cr = jnp.cumsum(hr)
    hit = cr >= want
    nset = jnp.sum(hit.astype(jnp.int32))
    idx = L - nset
    sel = iota == idx
    cr_at = jnp.sum(jnp.where(sel, cr, 0))
    h_at = jnp.sum(jnp.where(sel, hr, 0))
    B = c * L + (L - 1) - idx
    r = want - (cr_at - h_at)
    return B, r, h_at


def _scan_fine_chunk(hist, c, want):
    return _scan_fine_h(hist[pl.ds(c * L, L)], c, want)


def _scan_hist_desc(hist, cstart, want):
    def cond(carry):
        c, acc, done = carry
        return jnp.logical_not(done) & (c >= 0)

    def body(carry):
        c, acc, done = carry
        tot = jnp.sum(hist[pl.ds(c * L, L)])
        found = (acc + tot) >= want
        return (c - 1, acc + jnp.where(found, 0, tot), done | found)

    c_end, acc, _ = lax.while_loop(
        cond, body, (cstart, jnp.int32(0), jnp.bool_(False))
    )
    return _scan_fine_chunk(hist, c_end + 1, want - acc)


def _scan_hist4_desc(hist, cstart, want):
    def cond(carry):
        c, acc, done = carry
        return jnp.logical_not(done) & (c >= 0)

    def body(carry):
        c, acc, done = carry
        tot = jnp.sum(_h4(hist, c))
        found = (acc + tot) >= want
        return (c - 1, acc + jnp.where(found, 0, tot), done | found)

    c_end, acc, _ = lax.while_loop(
        cond, body, (cstart, jnp.int32(0), jnp.bool_(False))
    )
    c_fin = c_end + 1
    return _scan_fine_h(_h4(hist, c_fin), c_fin, want - acc)


def _zero(ref, n):
    zeros_i = jnp.zeros((L,), jnp.int32)

    @plsc.parallel_loop(0, n, step=L * 4)
    def _(i):
        for u in range(4):
            ref[pl.ds(i + u * L, L)] = zeros_i


def _exact_select(x_hbm, xv, hist, row, prefix, r2):
    ones = jnp.ones((L,), jnp.int32)
    pltpu.sync_copy(x_hbm.at[row], xv)
    _zero(hist, NB3)

    @pl.loop(0, N, step=L)
    def _(i):
        _, v = _remap_chunk(xv, i)
        m = (v >> 8) == (prefix >> 8)
        plsc.addupdate_scatter(hist, [v & 0xFF], ones, mask=m)

    B3, r3, neq = _scan_hist_desc(hist, jnp.int32(NB3 // L - 1), r2)
    vt = prefix + B3
    surplus = neq - r3

    @pl.when(surplus > 0)
    def _():
        def body(c, left):
            cc = NCHUNK - 1 - c
            xx, v = _remap_chunk(xv, cc * L)
            er = lax.rev((v == vt).astype(jnp.int32), (0,))
            pc = jnp.cumsum(er)
            zr = er * (pc <= left).astype(jnp.int32)
            zmask = lax.rev(zr, (0,)) > 0
            xv[pl.ds(cc * L, L)] = jnp.where(zmask, jnp.float32(0.0), xx)
            return left - jnp.sum(zr)

        lax.fori_loop(0, NCHUNK, body, surplus)

    @pl.loop(0, N, step=L)
    def _(i):
        xx, v = _remap_chunk(xv, i)
        xv[pl.ds(i, L)] = jnp.where(v >= vt, xx, jnp.float32(0.0))


def _do_row(x_hbm, xv, hist, coarse, row):
    ones = jnp.ones((L,), jnp.int32)

    _zero(hist, NSUB * NB)
    laneoff = (lax.iota(jnp.int32, L) & (NSUB - 1)) << 12

    @plsc.parallel_loop(0, N, step=L * UNROLL,
                        carry=jnp.full((L,), INT_MIN, jnp.int32))
    def _p1(i, vmax):
        for u in range(UNROLL):
            _, v = _remap_chunk(xv, i + u * L)
            plsc.addupdate_scatter(hist, [((v >> 20) + 2048) + laneoff], ones)
            vmax = jnp.maximum(vmax, v)
        return vmax

    bmax = (jnp.max(_p1) >> 20) + 2048
    B1, r1, _n1 = _scan_hist4_desc(hist, bmax >> 4, jnp.int32(TOPK))

    _zero(hist, NB)
    _zero(coarse, NC)

    @plsc.parallel_loop(0, N, step=L * UNROLL)
    def _(i):
        for u in range(UNROLL):
            _, v = _remap_chunk(xv, i + u * L)
            m = (v >> 20) == (B1 - 2048)
            b2 = (v >> 8) & 0xFFF
            plsc.addupdate_scatter(hist, [b2], ones, mask=m)
            plsc.addupdate_scatter(coarse, [b2 >> 4], ones, mask=m)

    C2, rC2, _ = _scan_hist_desc(coarse, jnp.int32(NC // L - 1), r1)
    B2, r2, n2 = _scan_fine_chunk(hist, C2, rC2)
    prefix = ((B1 - 2048) << 20) + (B2 << 8)
    surplus24 = n2 - r2

    @pl.when(surplus24 == 0)
    def _():
        @plsc.parallel_loop(0, N, step=L * UNROLL)
        def _(i):
            for u in range(UNROLL):
                xx, v = _remap_chunk(xv, i + u * L)
                xv[pl.ds(i + u * L, L)] = jnp.where(
                    v >= prefix, xx, jnp.float32(0.0)
                )

    @pl.when(surplus24 > 0)
    def _():
        _exact_select(x_hbm, xv, hist, row, prefix, r2)


def kernel(x):
    mesh = plsc.VectorSubcoreMesh(core_axis_name="c", subcore_axis_name="s")
    cp = pltpu.CompilerParams()
    if "needs_layout_passes" in pltpu.CompilerParams.__dataclass_fields__:
        cp = dataclasses.replace(cp, needs_layout_passes=False)

    @functools.partial(
        pl.kernel,
        out_type=jax.ShapeDtypeStruct((ROWS, N), jnp.float32),
        mesh=mesh,
        compiler_params=cp,
        scratch_types=[
            pltpu.VMEM((N,), jnp.float32),
            pltpu.VMEM((N,), jnp.float32),
            pltpu.VMEM((NSUB * NB,), jnp.int32),
            pltpu.VMEM((NC,), jnp.int32),
            pltpu.SemaphoreType.DMA,
            pltpu.SemaphoreType.DMA,
            pltpu.SemaphoreType.DMA,
            pltpu.SemaphoreType.DMA,
        ],
    )
    def _topk_mask(x_hbm, o_hbm, buf0, buf1, hist, coarse,
                   sin0, sin1, sout0, sout1):
        wid = lax.axis_index("s") * 2 + lax.axis_index("c")
        r0 = wid * ROWS_PER_W
        r1_ = r0 + 1
        in0 = pltpu.make_async_copy(x_hbm.at[r0], buf0, sin0)
        in1 = pltpu.make_async_copy(x_hbm.at[r1_], buf1, sin1)
        in0.start()
        in1.start()
        in0.wait()
        _do_row(x_hbm, buf0, hist, coarse, r0)
        out0 = pltpu.make_async_copy(buf0, o_hbm.at[r0], sout0)
        out0.start()
        in1.wait()
        _do_row(x_hbm, buf1, hist, coarse, r1_)
        out1 = pltpu.make_async_copy(buf1, o_hbm.at[r1_], sout1)
        out1.start()
        out0.wait()
        out1.wait()

    return _topk_mask(x)


# speedup vs baseline: 1.0688x; 1.0688x over previous
import dataclasses
import functools

import jax
import jax.numpy as jnp
from jax import lax
from jax.experimental import pallas as pl
from jax.experimental.pallas import tpu as pltpu
from jax.experimental.pallas import tpu_sc as plsc

ROWS = 64
N = 32768
TOPK = 64
L = 16
NCHUNK = N // L
NB = 4096
NC = NB // L
NB3 = 256
NWORKERS = 32
ROWS_PER_W = ROWS // NWORKERS
UNROLL = 8
INT_MIN = -(2**31)


def _monotone(xi):
    return xi ^ (jnp.int32(0x7FFFFFFF) & (xi >> 31))


def _unmap_f32(v):
    bits = jnp.where(v >= 0, v, v ^ jnp.int32(0x7FFFFFFF))
    return plsc.bitcast(jnp.full((L,), 0, jnp.int32) + bits, jnp.float32)


def _remap_chunk(xv, i):
    xx = xv[pl.ds(i, L)]
    return xx, _monotone(plsc.bitcast(xx, jnp.int32))


def _scan_fine_chunk(hist, c, want):
    iota = lax.iota(jnp.int32, L)
    h = hist[pl.ds(c * L, L)]
    hr = lax.rev(h, (0,))
    cr = jnp.cumsum(hr)
    hit = cr >= want
    nset = jnp.sum(hit.astype(jnp.int32))
    idx = L - nset
    sel = iota == idx
    cr_at = jnp.sum(jnp.where(sel, cr, 0))
    h_at = jnp.sum(jnp.where(sel, hr, 0))
    B = c * L + (L - 1) - idx
    r = want - (cr_at - h_at)
    return B, r, h_at


def _scan_hist_desc(hist, cstart, want):
    def cond(carry):
        c, acc, done = carry
        return jnp.logical_not(done) & (c >= 0)

    def body(carry):
        c, acc, done = carry
        tot = jnp.sum(hist[pl.ds(c * L, L)])
        found = (acc + tot) >= want
        return (c - 1, acc + jnp.where(found, 0, tot), done | found)

    c_end, acc, _ = lax.while_loop(
        cond, body, (cstart, jnp.int32(0), jnp.bool_(False))
    )
    return _scan_fine_chunk(hist, c_end + 1, want - acc)


def _zero(ref, n):
    zeros_i = jnp.zeros((L,), jnp.int32)

    @plsc.parallel_loop(0, n, step=L * 4)
    def _(i):
        for u in range(4):
            ref[pl.ds(i + u * L, L)] = zeros_i


def _exact_select(x_hbm, xv, hist, row, prefix, r2):
    ones = jnp.ones((L,), jnp.int32)
    pltpu.sync_copy(x_hbm.at[row], xv)
    _zero(hist, NB3)

    @pl.loop(0, N, step=L)
    def _(i):
        _, v = _remap_chunk(xv, i)
        m = (v >> 8) == (prefix >> 8)
        plsc.addupdate_scatter(hist, [v & 0xFF], ones, mask=m)

    B3, r3, neq = _scan_hist_desc(hist, jnp.int32(NB3 // L - 1), r2)
    vt = prefix + B3
    surplus = neq - r3

    @pl.when(surplus > 0)
    def _():
        def body(c, left):
            cc = NCHUNK - 1 - c
            xx, v = _remap_chunk(xv, cc * L)
            er = lax.rev((v == vt).astype(jnp.int32), (0,))
            pc = jnp.cumsum(er)
            zr = er * (pc <= left).astype(jnp.int32)
            zmask = lax.rev(zr, (0,)) > 0
            xv[pl.ds(cc * L, L)] = jnp.where(zmask, jnp.float32(0.0), xx)
            return left - jnp.sum(zr)

        lax.fori_loop(0, NCHUNK, body, surplus)

    @pl.loop(0, N, step=L)
    def _(i):
        xx, v = _remap_chunk(xv, i)
        xv[pl.ds(i, L)] = jnp.where(v >= vt, xx, jnp.float32(0.0))


def _do_row_common(xv, hist, coarse):
    ones = jnp.ones((L,), jnp.int32)

    _zero(hist, NB)

    @plsc.parallel_loop(0, N, step=L * UNROLL,
                        carry=jnp.full((L,), INT_MIN, jnp.int32))
    def _p1(i, vmax):
        for u in range(UNROLL):
            _, v = _remap_chunk(xv, i + u * L)
            plsc.addupdate_scatter(hist, [(v >> 20) + 2048], ones)
            vmax = jnp.maximum(vmax, v)
        return vmax

    bmax = (jnp.max(_p1) >> 20) + 2048
    B1, r1, _n1 = _scan_hist_desc(hist, bmax >> 4, jnp.int32(TOPK))

    _zero(hist, NB)
    _zero(coarse, NC)
    sub2 = (B1 - 2048) << 12

    @plsc.parallel_loop(0, N, step=L * UNROLL)
    def _(i):
        for u in range(UNROLL):
            _, v = _remap_chunk(xv, i + u * L)
            b2 = (v >> 8) - sub2
            m = plsc.bitcast(b2, jnp.uint32) < jnp.uint32(NB)
            plsc.addupdate_scatter(hist, [b2], ones, mask=m)
            plsc.addupdate_scatter(coarse, [b2 >> 4], ones, mask=m)

    C2, rC2, _ = _scan_hist_desc(coarse, jnp.int32(NC // L - 1), r1)
    B2, r2, n2 = _scan_fine_chunk(hist, C2, rC2)
    prefix = ((B1 - 2048) << 20) + (B2 << 8)
    surplus24 = n2 - r2

    @pl.when(surplus24 == 0)
    def _():
        tf = _unmap_f32(prefix)

        @plsc.parallel_loop(0, N, step=L * UNROLL)
        def _(i):
            for u in range(UNROLL):
                xx = xv[pl.ds(i + u * L, L)]
                xv[pl.ds(i + u * L, L)] = jnp.where(
                    xx >= tf, xx, jnp.float32(0.0)
                )

    return prefix, r2, surplus24


def kernel(x):
    mesh = plsc.VectorSubcoreMesh(core_axis_name="c", subcore_axis_name="s")
    cp = pltpu.CompilerParams()
    if "needs_layout_passes" in pltpu.CompilerParams.__dataclass_fields__:
        cp = dataclasses.replace(cp, needs_layout_passes=False)

    @functools.partial(
        pl.kernel,
        out_type=jax.ShapeDtypeStruct((ROWS, N), jnp.float32),
        mesh=mesh,
        compiler_params=cp,
        scratch_types=[
            pltpu.VMEM((N,), jnp.float32),
            pltpu.VMEM((N,), jnp.float32),
            pltpu.VMEM((NB,), jnp.int32),
            pltpu.VMEM((NC,), jnp.int32),
            pltpu.SemaphoreType.DMA,
            pltpu.SemaphoreType.DMA,
            pltpu.SemaphoreType.DMA,
            pltpu.SemaphoreType.DMA,
        ],
    )
    def _topk_mask(x_hbm, o_hbm, buf0, buf1, hist, coarse,
                   sin0, sin1, sout0, sout1):
        wid = lax.axis_index("s") * 2 + lax.axis_index("c")
        r0 = wid * ROWS_PER_W
        r1_ = r0 + 1
        in0 = pltpu.make_async_copy(x_hbm.at[r0], buf0, sin0)
        in1 = pltpu.make_async_copy(x_hbm.at[r1_], buf1, sin1)
        in0.start()
        in1.start()
        in0.wait()
        pfx0, rr0, sp0 = _do_row_common(buf0, hist, coarse)
        out0 = pltpu.make_async_copy(buf0, o_hbm.at[r0], sout0)
        out0.start()
        in1.wait()
        pfx1, rr1, sp1 = _do_row_common(buf1, hist, coarse)
        out1 = pltpu.make_async_copy(buf1, o_hbm.at[r1_], sout1)
        out1.start()
        out0.wait()
        out1.wait()

        @pl.loop(0, ROWS_PER_W)
        def _(j):
            surplus = jnp.where(j == 0, sp0, sp1)

            @pl.when(surplus > 0)
            def _():
                row = r0 + j
                prefix = jnp.where(j == 0, pfx0, pfx1)
                r2 = jnp.where(j == 0, rr0, rr1)
                _exact_select(x_hbm, buf0, hist, row, prefix, r2)
                pltpu.sync_copy(buf0, o_hbm.at[row])

    return _topk_mask(x)


# speedup vs baseline: 1.0991x; 1.0283x over previous
import dataclasses
import functools

import jax
import jax.numpy as jnp
from jax import lax
from jax.experimental import pallas as pl
from jax.experimental.pallas import tpu as pltpu
from jax.experimental.pallas import tpu_sc as plsc

ROWS = 64
N = 32768
TOPK = 64
L = 16
NCHUNK = N // L
NB = 4096
NC = NB // L
NB3 = 256
NWORKERS = 32
ROWS_PER_W = ROWS // NWORKERS
UNROLL = 8
INT_MIN = -(2**31)


def _monotone(xi):
    return xi ^ (jnp.int32(0x7FFFFFFF) & (xi >> 31))


def _unmap_f32(v):
    bits = jnp.where(v >= 0, v, v ^ jnp.int32(0x7FFFFFFF))
    return plsc.bitcast(jnp.full((L,), 0, jnp.int32) + bits, jnp.float32)


def _remap_chunk(xv, i):
    xx = xv[pl.ds(i, L)]
    return xx, _monotone(plsc.bitcast(xx, jnp.int32))


def _scan_fine_chunk(hist, c, want):
    iota = lax.iota(jnp.int32, L)
    h = hist[pl.ds(c * L, L)]
    hr = lax.rev(h, (0,))
    cr = jnp.cumsum(hr)
    hit = cr >= want
    nset = jnp.sum(hit.astype(jnp.int32))
    idx = L - nset
    sel = iota == idx
    cr_at = jnp.sum(jnp.where(sel, cr, 0))
    h_at = jnp.sum(jnp.where(sel, hr, 0))
    B = c * L + (L - 1) - idx
    r = want - (cr_at - h_at)
    return B, r, h_at


def _scan_hist_desc(hist, cstart, want):
    def cond(carry):
        c, acc, done = carry
        return jnp.logical_not(done) & (c >= 0)

    def body(carry):
        c, acc, done = carry
        tot = jnp.sum(hist[pl.ds(c * L, L)])
        found = (acc + tot) >= want
        return (c - 1, acc + jnp.where(found, 0, tot), done | found)

    c_end, acc, _ = lax.while_loop(
        cond, body, (cstart, jnp.int32(0), jnp.bool_(False))
    )
    return _scan_fine_chunk(hist, c_end + 1, want - acc)


def _zero(ref, n):
    zeros_i = jnp.zeros((L,), jnp.int32)

    @plsc.parallel_loop(0, n, step=L * 4)
    def _(i):
        for u in range(4):
            ref[pl.ds(i + u * L, L)] = zeros_i


def _exact_select(x_hbm, xv, base, hist, row, prefix, r2):
    ones = jnp.ones((L,), jnp.int32)
    pltpu.sync_copy(x_hbm.at[row], xv.at[pl.ds(base, N)])
    _zero(hist, NB3)

    @pl.loop(0, N, step=L)
    def _(i):
        _, v = _remap_chunk(xv, base + i)
        m = (v >> 8) == (prefix >> 8)
        plsc.addupdate_scatter(hist, [v & 0xFF], ones, mask=m)

    B3, r3, neq = _scan_hist_desc(hist, jnp.int32(NB3 // L - 1), r2)
    vt = prefix + B3
    surplus = neq - r3

    @pl.when(surplus > 0)
    def _():
        def body(c, left):
            cc = base + (NCHUNK - 1 - c) * L
            xx, v = _remap_chunk(xv, cc)
            er = lax.rev((v == vt).astype(jnp.int32), (0,))
            pc = jnp.cumsum(er)
            zr = er * (pc <= left).astype(jnp.int32)
            zmask = lax.rev(zr, (0,)) > 0
            xv[pl.ds(cc, L)] = jnp.where(zmask, jnp.float32(0.0), xx)
            return left - jnp.sum(zr)

        lax.fori_loop(0, NCHUNK, body, surplus)

    @pl.loop(0, N, step=L)
    def _(i):
        xx, v = _remap_chunk(xv, base + i)
        xv[pl.ds(base + i, L)] = jnp.where(v >= vt, xx, jnp.float32(0.0))


def _do_row_common(xv, base, hist, coarse):
    ones = jnp.ones((L,), jnp.int32)

    _zero(hist, NB)

    @plsc.parallel_loop(0, N, step=L * UNROLL,
                        carry=jnp.full((L,), INT_MIN, jnp.int32))
    def _p1(i, vmax):
        for u in range(UNROLL):
            _, v = _remap_chunk(xv, base + i + u * L)
            plsc.addupdate_scatter(hist, [(v >> 20) + 2048], ones)
            vmax = jnp.maximum(vmax, v)
        return vmax

    bmax = (jnp.max(_p1) >> 20) + 2048
    B1, r1, _n1 = _scan_hist_desc(hist, bmax >> 4, jnp.int32(TOPK))

    _zero(hist, NB)
    _zero(coarse, NC)
    sub2 = (B1 - 2048) << 12

    @plsc.parallel_loop(0, N, step=L * UNROLL)
    def _(i):
        for u in range(UNROLL):
            _, v = _remap_chunk(xv, base + i + u * L)
            b2 = (v >> 8) - sub2
            m = plsc.bitcast(b2, jnp.uint32) < jnp.uint32(NB)
            plsc.addupdate_scatter(hist, [b2], ones, mask=m)
            plsc.addupdate_scatter(coarse, [b2 >> 4], ones, mask=m)

    C2, rC2, _ = _scan_hist_desc(coarse, jnp.int32(NC // L - 1), r1)
    B2, r2, n2 = _scan_fine_chunk(hist, C2, rC2)
    prefix = ((B1 - 2048) << 20) + (B2 << 8)
    surplus24 = n2 - r2

    @pl.when(surplus24 == 0)
    def _():
        tf = _unmap_f32(prefix)

        @plsc.parallel_loop(0, N, step=L * UNROLL)
        def _(i):
            for u in range(UNROLL):
                xx = xv[pl.ds(base + i + u * L, L)]
                xv[pl.ds(base + i + u * L, L)] = jnp.where(
                    xx >= tf, xx, jnp.float32(0.0)
                )

    return prefix, r2, surplus24


def kernel(x):
    mesh = plsc.VectorSubcoreMesh(core_axis_name="c", subcore_axis_name="s")
    cp = pltpu.CompilerParams()
    if "needs_layout_passes" in pltpu.CompilerParams.__dataclass_fields__:
        cp = dataclasses.replace(cp, needs_layout_passes=False)

    @functools.partial(
        pl.kernel,
        out_type=jax.ShapeDtypeStruct((ROWS, N), jnp.float32),
        mesh=mesh,
        compiler_params=cp,
        scratch_types=[
            pltpu.VMEM((ROWS_PER_W * N,), jnp.float32),
            pltpu.VMEM((NB,), jnp.int32),
            pltpu.VMEM((NC,), jnp.int32),
            pltpu.SemaphoreType.DMA,
            pltpu.SemaphoreType.DMA,
            pltpu.SemaphoreType.DMA,
            pltpu.SemaphoreType.DMA,
        ],
    )
    def _topk_mask(x_hbm, o_hbm, buf, hist, coarse,
                   sin0, sin1, sout0, sout1):
        wid = lax.axis_index("s") * 2 + lax.axis_index("c")
        r0 = wid * ROWS_PER_W
        in0 = pltpu.make_async_copy(x_hbm.at[r0], buf.at[pl.ds(0, N)], sin0)
        in1 = pltpu.make_async_copy(x_hbm.at[r0 + 1], buf.at[pl.ds(N, N)],
                                    sin1)
        out0 = pltpu.make_async_copy(buf.at[pl.ds(0, N)], o_hbm.at[r0],
                                     sout0)
        out1 = pltpu.make_async_copy(buf.at[pl.ds(N, N)], o_hbm.at[r0 + 1],
                                     sout1)
        in0.start()
        in1.start()

        @pl.loop(0, ROWS_PER_W)
        def _(j):
            row = r0 + j
            base = j * N

            @pl.when(j == 0)
            def _():
                in0.wait()

            @pl.when(j == 1)
            def _():
                in1.wait()

            prefix, r2, surplus = _do_row_common(buf, base, hist, coarse)

            @pl.when(surplus > 0)
            def _():
                _exact_select(x_hbm, buf, base, hist, row, prefix, r2)

            @pl.when(j == 0)
            def _():
                out0.start()

            @pl.when(j == 1)
            def _():
                out1.start()

        out0.wait()
        out1.wait()

    return _topk_mask(x)
